# Initial kernel scaffold; baseline (speedup 1.0000x reference)
import jax
import jax.numpy as jnp
from jax.experimental import pallas as pl


def kernel(objectness, pred_bbox_deltas, anchors):
    raise NotImplementedError("write your pallas kernel here")


# speedup vs baseline: 29.3390x; 29.3390x over previous
import functools

import jax
import jax.numpy as jnp
from jax.experimental import pallas as pl

_IMG_W = 800.0
_IMG_H = 800.0
_PRE_NMS_TOP_N = 2000
_POST_NMS_TOP_N = 1000
_NMS_THRESH = 0.7
_MIN_SIZE = 1.0
_A = 3
_BBOX_XFORM_CLIP = 4.135166556742356

_N_REAL = 19200
_ROWS = 152
_LANES = 128
_N_PAD = _ROWS * _LANES
_NEG = -3.0e38


def _nms_kernel(obj_ref, dx_ref, dy_ref, dw_ref, dh_ref,
                ax1_ref, ay1_ref, ax2_ref, ay2_ref,
                ox1_ref, oy1_ref, ox2_ref, oy2_ref, os_ref):
    f32 = jnp.float32
    obj = obj_ref[:]
    ax1 = ax1_ref[:]
    ay1 = ay1_ref[:]
    ax2 = ax2_ref[:]
    ay2 = ay2_ref[:]

    widths = ax2 - ax1
    heights = ay2 - ay1
    ctr_x = ax1 + 0.5 * widths
    ctr_y = ay1 + 0.5 * heights
    dw = jnp.minimum(dw_ref[:], f32(_BBOX_XFORM_CLIP))
    dh = jnp.minimum(dh_ref[:], f32(_BBOX_XFORM_CLIP))
    pcx = dx_ref[:] * widths + ctr_x
    pcy = dy_ref[:] * heights + ctr_y
    pw = jnp.exp(dw) * widths
    ph = jnp.exp(dh) * heights
    x1 = jnp.clip(pcx - 0.5 * pw, 0.0, _IMG_W)
    y1 = jnp.clip(pcy - 0.5 * ph, 0.0, _IMG_H)
    x2 = jnp.clip(pcx + 0.5 * pw, 0.0, _IMG_W)
    y2 = jnp.clip(pcy + 0.5 * ph, 0.0, _IMG_H)
    valid = ((x2 - x1) >= _MIN_SIZE) & ((y2 - y1) >= _MIN_SIZE)

    area = (x2 - x1) * (y2 - y1)
    score = jax.nn.sigmoid(obj)

    lin = (jax.lax.broadcasted_iota(jnp.int32, (_ROWS, _LANES), 0) * _LANES
           + jax.lax.broadcasted_iota(jnp.int32, (_ROWS, _LANES), 1))
    real = lin < _N_REAL

    u = jax.lax.bitcast_convert_type(obj, jnp.int32)
    r = jnp.where(u < 0, u ^ jnp.int32(0x7FFFFFFF), u)
    v = r ^ jnp.int32(-0x80000000)

    def bit_body(b, carry):
        prefix, greater = carry
        bit = jnp.int32(1) << b
        himask = jnp.int32(-1) << b
        trial = prefix | bit
        c = jnp.sum(jnp.where(real & (((v ^ trial) & himask) == 0),
                              jnp.int32(1), jnp.int32(0)))
        take = (greater + c) >= _PRE_NMS_TOP_N
        prefix = jnp.where(take, trial, prefix)
        greater = jnp.where(take, greater, greater + c)
        return prefix, greater

    prefix, greater = (jnp.int32(0), jnp.int32(0))
    prefix, greater = jax.lax.fori_loop(
        0, 32, lambda i, cg: bit_body(31 - i, cg), (prefix, greater))
    k_r = prefix ^ jnp.int32(-0x80000000)
    need = _PRE_NMS_TOP_N - greater

    eq_cut = real & (r == k_r)

    def idx_body(_, lohi):
        lo, hi = lohi
        mid = (lo + hi) // 2
        c = jnp.sum(jnp.where(eq_cut & (lin <= mid),
                              jnp.int32(1), jnp.int32(0)))
        ok = c >= need
        return jnp.where(ok, lo, mid), jnp.where(ok, mid, hi)

    lo, hi = jax.lax.fori_loop(0, 15, idx_body,
                               (jnp.int32(-1), jnp.int32(_N_PAD - 1)))
    idx_cut = hi

    selected = real & ((r > k_r) | ((r == k_r) & (lin <= idx_cut)))
    active0 = jnp.where(selected & valid, f32(1.0), f32(0.0))

    out_lane = (jax.lax.broadcasted_iota(jnp.int32, (8, _LANES), 0) * _LANES
                + jax.lax.broadcasted_iota(jnp.int32, (8, _LANES), 1))

    def nms_body(i, carry):
        active, o_x1, o_y1, o_x2, o_y2, o_s = carry
        key = jnp.where(active > 0.5, obj, f32(_NEG))
        m = jnp.max(key)
        found = m > f32(_NEG) * 0.5
        eq = key == m
        idxm = jnp.min(jnp.where(eq, lin, jnp.int32(0x3FFFFFFF)))
        sel1 = lin == idxm
        zero = f32(0.0)
        cx1 = jnp.sum(jnp.where(sel1, x1, zero))
        cy1 = jnp.sum(jnp.where(sel1, y1, zero))
        cx2 = jnp.sum(jnp.where(sel1, x2, zero))
        cy2 = jnp.sum(jnp.where(sel1, y2, zero))
        cs = jnp.sum(jnp.where(sel1, score, zero))
        cx1 = jnp.where(found, cx1, zero)
        cy1 = jnp.where(found, cy1, zero)
        cx2 = jnp.where(found, cx2, zero)
        cy2 = jnp.where(found, cy2, zero)
        cs = jnp.where(found, cs, zero)
        ca = (cx2 - cx1) * (cy2 - cy1)

        xx1 = jnp.maximum(cx1, x1)
        yy1 = jnp.maximum(cy1, y1)
        xx2 = jnp.minimum(cx2, x2)
        yy2 = jnp.minimum(cy2, y2)
        inter = (jnp.maximum(xx2 - xx1, 0.0) * jnp.maximum(yy2 - yy1, 0.0))
        iou = inter / (ca + area - inter + 1e-9)
        active = jnp.where((iou > _NMS_THRESH) | (lin == idxm), f32(0.0), active)

        put = out_lane == i
        o_x1 = jnp.where(put, cx1, o_x1)
        o_y1 = jnp.where(put, cy1, o_y1)
        o_x2 = jnp.where(put, cx2, o_x2)
        o_y2 = jnp.where(put, cy2, o_y2)
        o_s = jnp.where(put, cs, o_s)
        return active, o_x1, o_y1, o_x2, o_y2, o_s

    z8 = jnp.zeros((8, _LANES), f32)
    carry = (active0, z8, z8, z8, z8, z8)
    carry = jax.lax.fori_loop(0, _POST_NMS_TOP_N, nms_body, carry)
    _, o_x1, o_y1, o_x2, o_y2, o_s = carry

    ox1_ref[:] = o_x1
    oy1_ref[:] = o_y1
    ox2_ref[:] = o_x2
    oy2_ref[:] = o_y2
    os_ref[:] = o_s


@functools.partial(jax.jit, static_argnames=())
def _run(obj_flat, dx, dy, dw, dh, ax1, ay1, ax2, ay2):
    def pad2d(a):
        a = jnp.pad(a, (0, _N_PAD - _N_REAL))
        return a.reshape(_ROWS, _LANES)

    args = [pad2d(a) for a in (obj_flat, dx, dy, dw, dh, ax1, ay1, ax2, ay2)]
    out_shape = [jax.ShapeDtypeStruct((8, _LANES), jnp.float32)] * 5
    o_x1, o_y1, o_x2, o_y2, o_s = pl.pallas_call(
        _nms_kernel,
        out_shape=out_shape,
    )(*args)
    boxes = jnp.stack([o_x1.reshape(-1)[:_POST_NMS_TOP_N],
                       o_y1.reshape(-1)[:_POST_NMS_TOP_N],
                       o_x2.reshape(-1)[:_POST_NMS_TOP_N],
                       o_y2.reshape(-1)[:_POST_NMS_TOP_N]], axis=1)
    scores = o_s.reshape(-1)[:_POST_NMS_TOP_N]
    return boxes, scores


def kernel(objectness, pred_bbox_deltas, anchors):
    obj_flat = objectness.reshape(_A, 80, 80).transpose(1, 2, 0).reshape(-1)
    d = pred_bbox_deltas.reshape(_A, 4, 80, 80).transpose(2, 3, 0, 1)
    d = d.reshape(-1, 4)
    return _run(obj_flat, d[:, 0], d[:, 1], d[:, 2], d[:, 3],
                anchors[:, 0], anchors[:, 1], anchors[:, 2], anchors[:, 3])


# speedup vs baseline: 31.0998x; 1.0600x over previous
import functools

import jax
import jax.numpy as jnp
from jax.experimental import pallas as pl
from jax.experimental.pallas import tpu as pltpu

_IMG_W = 800.0
_IMG_H = 800.0
_PRE_NMS_TOP_N = 2000
_POST_NMS_TOP_N = 1000
_NMS_THRESH = 0.7
_MIN_SIZE = 1.0
_A = 3
_BBOX_XFORM_CLIP = 4.135166556742356

_N_REAL = 19200
_ROWS = 152
_LANES = 128
_N_PAD = _ROWS * _LANES
_NEG = -3.0e38


def _nms_kernel(obj_ref, dx_ref, dy_ref, dw_ref, dh_ref,
                ax1_ref, ay1_ref, ax2_ref, ay2_ref,
                ox1_ref, oy1_ref, ox2_ref, oy2_ref, os_ref,
                key_ref, x1_ref, y1_ref, x2_ref, y2_ref, ar_ref, sc_ref):
    f32 = jnp.float32
    obj = obj_ref[:]
    ax1 = ax1_ref[:]
    ay1 = ay1_ref[:]
    ax2 = ax2_ref[:]
    ay2 = ay2_ref[:]

    widths = ax2 - ax1
    heights = ay2 - ay1
    ctr_x = ax1 + 0.5 * widths
    ctr_y = ay1 + 0.5 * heights
    dw = jnp.minimum(dw_ref[:], f32(_BBOX_XFORM_CLIP))
    dh = jnp.minimum(dh_ref[:], f32(_BBOX_XFORM_CLIP))
    pcx = dx_ref[:] * widths + ctr_x
    pcy = dy_ref[:] * heights + ctr_y
    pw = jnp.exp(dw) * widths
    ph = jnp.exp(dh) * heights
    x1 = jnp.clip(pcx - 0.5 * pw, 0.0, _IMG_W)
    y1 = jnp.clip(pcy - 0.5 * ph, 0.0, _IMG_H)
    x2 = jnp.clip(pcx + 0.5 * pw, 0.0, _IMG_W)
    y2 = jnp.clip(pcy + 0.5 * ph, 0.0, _IMG_H)
    valid = ((x2 - x1) >= _MIN_SIZE) & ((y2 - y1) >= _MIN_SIZE)

    lin = (jax.lax.broadcasted_iota(jnp.int32, (_ROWS, _LANES), 0) * _LANES
           + jax.lax.broadcasted_iota(jnp.int32, (_ROWS, _LANES), 1))
    real = lin < _N_REAL

    u = jax.lax.bitcast_convert_type(obj, jnp.int32)
    r = jnp.where(u < 0, u ^ jnp.int32(0x7FFFFFFF), u)
    v = r ^ jnp.int32(-0x80000000)

    def bit_body(b, carry):
        prefix, greater = carry
        bit = jnp.int32(1) << b
        himask = jnp.int32(-1) << b
        trial = prefix | bit
        c = jnp.sum(jnp.where(real & (((v ^ trial) & himask) == 0),
                              jnp.int32(1), jnp.int32(0)))
        take = (greater + c) >= _PRE_NMS_TOP_N
        prefix = jnp.where(take, trial, prefix)
        greater = jnp.where(take, greater, greater + c)
        return prefix, greater

    prefix, greater = (jnp.int32(0), jnp.int32(0))
    prefix, greater = jax.lax.fori_loop(
        0, 32, lambda i, cg: bit_body(31 - i, cg), (prefix, greater))
    k_r = prefix ^ jnp.int32(-0x80000000)
    need = _PRE_NMS_TOP_N - greater

    eq_cut = real & (r == k_r)

    def idx_body(_, lohi):
        lo, hi = lohi
        mid = (lo + hi) // 2
        c = jnp.sum(jnp.where(eq_cut & (lin <= mid),
                              jnp.int32(1), jnp.int32(0)))
        ok = c >= need
        return jnp.where(ok, lo, mid), jnp.where(ok, mid, hi)

    lo, hi = jax.lax.fori_loop(0, 15, idx_body,
                               (jnp.int32(-1), jnp.int32(_N_PAD - 1)))
    idx_cut = hi

    selected = real & ((r > k_r) | ((r == k_r) & (lin <= idx_cut)))

    key_ref[:] = jnp.where(selected & valid, obj, f32(_NEG))
    x1_ref[:] = x1
    y1_ref[:] = y1
    x2_ref[:] = x2
    y2_ref[:] = y2
    ar_ref[:] = (x2 - x1) * (y2 - y1)
    sc_ref[:] = jax.nn.sigmoid(obj)

    out_lane = (jax.lax.broadcasted_iota(jnp.int32, (8, _LANES), 0) * _LANES
                + jax.lax.broadcasted_iota(jnp.int32, (8, _LANES), 1))
    lane1 = jax.lax.broadcasted_iota(jnp.int32, (1, _LANES), 1)

    def pick(ref, row, colm):
        rowv = ref[pl.ds(row, 1), :]
        return jnp.sum(jnp.where(colm, rowv, f32(0.0)))

    def nms_body(i, carry):
        o_x1, o_y1, o_x2, o_y2, o_s = carry
        key = key_ref[:]
        m = jnp.max(key)
        eq = key == m
        idxm = jnp.min(jnp.where(eq, lin, jnp.int32(0x3FFFFFFF)))
        row = idxm >> 7
        colm = lane1 == (idxm & 127)
        found = m > f32(_NEG) * 0.5
        zero = f32(0.0)
        cx1 = jnp.where(found, pick(x1_ref, row, colm), zero)
        cy1 = jnp.where(found, pick(y1_ref, row, colm), zero)
        cx2 = jnp.where(found, pick(x2_ref, row, colm), zero)
        cy2 = jnp.where(found, pick(y2_ref, row, colm), zero)
        cs = jnp.where(found, pick(sc_ref, row, colm), zero)
        ca = (cx2 - cx1) * (cy2 - cy1)

        xx1 = jnp.maximum(cx1, x1_ref[:])
        yy1 = jnp.maximum(cy1, y1_ref[:])
        xx2 = jnp.minimum(cx2, x2_ref[:])
        yy2 = jnp.minimum(cy2, y2_ref[:])
        inter = (jnp.maximum(xx2 - xx1, 0.0) * jnp.maximum(yy2 - yy1, 0.0))
        iou = inter / (ca + ar_ref[:] - inter + 1e-9)
        key_ref[:] = jnp.where((iou > _NMS_THRESH) | (lin == idxm),
                               f32(_NEG), key)

        put = out_lane == i
        o_x1 = jnp.where(put, cx1, o_x1)
        o_y1 = jnp.where(put, cy1, o_y1)
        o_x2 = jnp.where(put, cx2, o_x2)
        o_y2 = jnp.where(put, cy2, o_y2)
        o_s = jnp.where(put, cs, o_s)
        return o_x1, o_y1, o_x2, o_y2, o_s

    z8 = jnp.zeros((8, _LANES), f32)
    carry = (z8, z8, z8, z8, z8)
    o_x1, o_y1, o_x2, o_y2, o_s = jax.lax.fori_loop(
        0, _POST_NMS_TOP_N, nms_body, carry)

    ox1_ref[:] = o_x1
    oy1_ref[:] = o_y1
    ox2_ref[:] = o_x2
    oy2_ref[:] = o_y2
    os_ref[:] = o_s


@functools.partial(jax.jit, static_argnames=())
def _run(obj_flat, dx, dy, dw, dh, ax1, ay1, ax2, ay2):
    def pad2d(a):
        a = jnp.pad(a, (0, _N_PAD - _N_REAL))
        return a.reshape(_ROWS, _LANES)

    args = [pad2d(a) for a in (obj_flat, dx, dy, dw, dh, ax1, ay1, ax2, ay2)]
    out_shape = [jax.ShapeDtypeStruct((8, _LANES), jnp.float32)] * 5
    o_x1, o_y1, o_x2, o_y2, o_s = pl.pallas_call(
        _nms_kernel,
        out_shape=out_shape,
        scratch_shapes=[pltpu.VMEM((_ROWS, _LANES), jnp.float32)] * 7,
    )(*args)
    boxes = jnp.stack([o_x1.reshape(-1)[:_POST_NMS_TOP_N],
                       o_y1.reshape(-1)[:_POST_NMS_TOP_N],
                       o_x2.reshape(-1)[:_POST_NMS_TOP_N],
                       o_y2.reshape(-1)[:_POST_NMS_TOP_N]], axis=1)
    scores = o_s.reshape(-1)[:_POST_NMS_TOP_N]
    return boxes, scores


def kernel(objectness, pred_bbox_deltas, anchors):
    obj_flat = objectness.reshape(_A, 80, 80).transpose(1, 2, 0).reshape(-1)
    d = pred_bbox_deltas.reshape(_A, 4, 80, 80).transpose(2, 3, 0, 1)
    d = d.reshape(-1, 4)
    return _run(obj_flat, d[:, 0], d[:, 1], d[:, 2], d[:, 3],
                anchors[:, 0], anchors[:, 1], anchors[:, 2], anchors[:, 3])
